# baseline (speedup 1.0000x reference)
import functools

import jax
import jax.numpy as jnp
from jax import lax
from jax.experimental import pallas as pl
from jax.experimental.pallas import tpu as pltpu
from jax.experimental.pallas import tpu_sc as plsc

_VOCAB, _EMB, _ASP = 1000, 64, 9
_B, _L = 4096, 200

_INFO = plsc.get_sparse_core_info()
_NC, _NS = _INFO.num_cores, _INFO.num_subcores
_NW = _NC * _NS
_LANES = 16
_BPW = _B // _NW
_LCH = 8
_NCHUNK = _L // _LCH
_NBG = _BPW // _LANES
_NPAIR = (_ASP + 1) // 2


def _proj_body(embT_ref, wT_ref, b_ref, out_ref):
    projT = jax.lax.dot_general(
        wT_ref[...], embT_ref[...], (((1,), (0,)), ((), ())),
        preferred_element_type=jnp.float32,
    ) + jnp.transpose(b_ref[...])
    u32 = jax.lax.bitcast_convert_type(
        projT.astype(jnp.bfloat16), jnp.uint16
    ).astype(jnp.uint32)
    rows = [u32[2 * p:2 * p + 1, :] | (u32[2 * p + 1:2 * p + 2, :] << 16)
            for p in range(4)]
    rows.append(u32[8:9, :])
    packed = jax.lax.bitcast_convert_type(
        jnp.concatenate(rows, axis=0), jnp.int32
    )
    out_ref[pl.ds(0, _NPAIR), pl.ds(0, _VOCAB)] = packed


def _gather_body(proj_hbm, wordsT_hbm, out_hbm,
                 proj_v, idx_v, out_a, out_b, sem_a, sem_b):
    wid = lax.axis_index("s") * _NC + lax.axis_index("c")
    b0 = wid * _BPW

    pltpu.sync_copy(proj_hbm, proj_v)
    pltpu.sync_copy(wordsT_hbm.at[:, pl.ds(b0, _BPW)], idx_v)

    def compute_chunk(lc, outv):
        l0 = lc * _LCH

        @plsc.parallel_loop(0, _NBG, unroll=8)
        def _(bg):
            for l in range(_LCH):
                tok = idx_v[l0 + l, pl.ds(bg * _LANES, _LANES)]
                for p in range(_NPAIR):
                    pv = plsc.load_gather(
                        proj_v, [jnp.full((_LANES,), p, jnp.int32), tok]
                    )
                    lo, hi = plsc.unpack(
                        plsc.bitcast(pv, jnp.bfloat16),
                        format=plsc.PackFormat.INTERLEAVED,
                        preferred_element_type=jnp.float32,
                    )
                    outv[2 * p, l, pl.ds(bg * _LANES, _LANES)] = lo
                    if p < _NPAIR - 1:
                        outv[2 * p + 1, l, pl.ds(bg * _LANES, _LANES)] = hi

    def store_chunk(lc, outv, sem):
        pltpu.async_copy(
            outv, out_hbm.at[:, pl.ds(lc * _LCH, _LCH), pl.ds(b0, _BPW)], sem
        )

    def drain(outv, sem):
        pltpu.make_async_copy(
            out_hbm.at[:, pl.ds(0, _LCH), pl.ds(0, _BPW)], outv, sem
        ).wait()

    def outer(p, carry):
        for par in range(2):
            lc = p * 2 + par
            outv = out_a if par == 0 else out_b
            sem = sem_a if par == 0 else sem_b

            @pl.when(p > 0)
            def _():
                drain(outv, sem)

            compute_chunk(lc, outv)
            store_chunk(lc, outv, sem)
        return carry

    lax.fori_loop(0, (_NCHUNK - 1) // 2, outer, 0)
    drain(out_a, sem_a)
    compute_chunk(jnp.int32(_NCHUNK - 1), out_a)
    store_chunk(jnp.int32(_NCHUNK - 1), out_a, sem_a)
    drain(out_a, sem_a)
    drain(out_b, sem_b)


_gather = functools.partial(
    pl.kernel,
    out_type=jax.ShapeDtypeStruct((_ASP, _L, _B), jnp.float32),
    mesh=plsc.VectorSubcoreMesh(core_axis_name="c", subcore_axis_name="s"),
    compiler_params=pltpu.CompilerParams(needs_layout_passes=False),
    scratch_types=[
        pltpu.VMEM((8, 1024), jnp.int32),
        pltpu.VMEM((_L, _BPW), jnp.int32),
        pltpu.VMEM((_ASP, _LCH, _BPW), jnp.float32),
        pltpu.VMEM((_ASP, _LCH, _BPW), jnp.float32),
        pltpu.SemaphoreType.DMA,
        pltpu.SemaphoreType.DMA,
    ],
)(_gather_body)


def kernel(words, emb_table, W, b):
    projT = pl.pallas_call(
        _proj_body,
        out_shape=jax.ShapeDtypeStruct((8, 1024), jnp.int32),
    )(jnp.transpose(emb_table), jnp.transpose(W), b.reshape(1, _ASP))
    out_t = _gather(projT, jnp.transpose(words))
    return jnp.transpose(out_t, (2, 1, 0))


# speedup vs baseline: 1.4649x; 1.4649x over previous
import functools

import jax
import jax.numpy as jnp
from jax import lax
from jax.experimental import pallas as pl
from jax.experimental.pallas import tpu as pltpu
from jax.experimental.pallas import tpu_sc as plsc

_VOCAB, _EMB, _ASP = 1000, 64, 9
_B, _L = 4096, 200

_INFO = plsc.get_sparse_core_info()
_NC, _NS = _INFO.num_cores, _INFO.num_subcores
_NW = _NC * _NS
_LANES = 16
_BPW = _B // _NW
_LCH = 8
_NCHUNK = _L // _LCH
_NBG = _BPW // _LANES
_NPAIR = (_ASP + 1) // 2


def _proj_body(embT_ref, wT_ref, b_ref, out_ref):
    projT = jax.lax.dot_general(
        wT_ref[...], embT_ref[...], (((1,), (0,)), ((), ())),
        preferred_element_type=jnp.float32,
    ) + jnp.transpose(b_ref[...])
    u32 = jax.lax.bitcast_convert_type(
        projT.astype(jnp.bfloat16), jnp.uint16
    ).astype(jnp.uint32)
    rows = [u32[2 * p:2 * p + 1, :] | (u32[2 * p + 1:2 * p + 2, :] << 16)
            for p in range(4)]
    rows.append(u32[8:9, :])
    packed = jax.lax.bitcast_convert_type(
        jnp.concatenate(rows, axis=0), jnp.int32
    )
    out_ref[pl.ds(0, _NPAIR), pl.ds(0, _VOCAB)] = packed


def _gather_body(proj_hbm, wordsT_hbm, out_hbm,
                 proj_v, idx_v, out_a, out_b, sem_a, sem_b):
    wid = lax.axis_index("s") * _NC + lax.axis_index("c")
    b0 = wid * _BPW

    pltpu.sync_copy(proj_hbm, proj_v)
    pltpu.sync_copy(wordsT_hbm.at[:, pl.ds(b0, _BPW)], idx_v)

    def compute_chunk(lc, outv):
        l0 = lc * _LCH

        @plsc.parallel_loop(0, _NBG, unroll=4)
        def _(bg):
            for l in range(_LCH):
                tok = idx_v[l0 + l, pl.ds(bg * _LANES, _LANES)]
                for p in range(_NPAIR):
                    pv = plsc.load_gather(
                        proj_v, [jnp.full((_LANES,), p, jnp.int32), tok]
                    )
                    lo, hi = plsc.unpack(
                        plsc.bitcast(pv, jnp.bfloat16),
                        format=plsc.PackFormat.INTERLEAVED,
                        preferred_element_type=jnp.float32,
                    )
                    outv[2 * p, l, pl.ds(bg * _LANES, _LANES)] = lo
                    if p < _NPAIR - 1:
                        outv[2 * p + 1, l, pl.ds(bg * _LANES, _LANES)] = hi

    def store_chunk(lc, outv, sem):
        pltpu.async_copy(
            outv, out_hbm.at[:, pl.ds(lc * _LCH, _LCH), pl.ds(b0, _BPW)], sem
        )

    def drain(outv, sem):
        pltpu.make_async_copy(
            out_hbm.at[:, pl.ds(0, _LCH), pl.ds(0, _BPW)], outv, sem
        ).wait()

    def outer(p, carry):
        for par in range(2):
            lc = p * 2 + par
            outv = out_a if par == 0 else out_b
            sem = sem_a if par == 0 else sem_b

            @pl.when(p > 0)
            def _():
                drain(outv, sem)

            compute_chunk(lc, outv)
            store_chunk(lc, outv, sem)
        return carry

    lax.fori_loop(0, (_NCHUNK - 1) // 2, outer, 0)
    drain(out_a, sem_a)
    compute_chunk(jnp.int32(_NCHUNK - 1), out_a)
    store_chunk(jnp.int32(_NCHUNK - 1), out_a, sem_a)
    drain(out_a, sem_a)
    drain(out_b, sem_b)


_gather = functools.partial(
    pl.kernel,
    out_type=jax.ShapeDtypeStruct((_ASP, _L, _B), jnp.float32),
    mesh=plsc.VectorSubcoreMesh(core_axis_name="c", subcore_axis_name="s"),
    compiler_params=pltpu.CompilerParams(needs_layout_passes=False),
    scratch_types=[
        pltpu.VMEM((8, 1024), jnp.int32),
        pltpu.VMEM((_L, _BPW), jnp.int32),
        pltpu.VMEM((_ASP, _LCH, _BPW), jnp.float32),
        pltpu.VMEM((_ASP, _LCH, _BPW), jnp.float32),
        pltpu.SemaphoreType.DMA,
        pltpu.SemaphoreType.DMA,
    ],
)(_gather_body)


def kernel(words, emb_table, W, b):
    projT = pl.pallas_call(
        _proj_body,
        out_shape=jax.ShapeDtypeStruct((8, 1024), jnp.int32),
    )(jnp.transpose(emb_table), jnp.transpose(W), b.reshape(1, _ASP))
    out_t = _gather(projT, jnp.transpose(words))
    return jnp.transpose(out_t, (2, 1, 0))
